# baseline (speedup 1.0000x reference)
import jax
import jax.numpy as jnp
from jax.experimental import pallas as pl
from jax.experimental.pallas import tpu as pltpu
from jax.experimental.pallas import tpu_sc as plsc

_RB = 4
_L = 16
_TC_BS = 256
_SC_BATCHES = 1


def _sc_part(x, pe_table):
    Bp, S, H = x.shape
    mesh = plsc.VectorSubcoreMesh(core_axis_name="c", subcore_axis_name="s")

    @pl.kernel(out_type=jax.ShapeDtypeStruct((Bp, S, H), x.dtype), mesh=mesh)
    def pe_add_sc(x_hbm, pe_hbm, o_hbm):
        def body(x_vmem, pe_vmem, o_vmem):
            for r in range(_RB):

                @plsc.parallel_loop(0, H, step=_L, unroll=4)
                def _chunk(col, _r=r):
                    slc = pl.ds(col, _L)
                    pe_chunk = pe_vmem.at[_r].at[slc][...]
                    for b in range(Bp):
                        o_vmem.at[b].at[_r].at[slc][...] = (
                            x_vmem.at[b].at[_r].at[slc][...] + pe_chunk
                        )

        pltpu.emit_pipeline(
            body,
            grid=(S // _RB,),
            in_specs=[
                pl.BlockSpec((Bp, _RB, H), lambda i: (0, i, 0)),
                pl.BlockSpec((_RB, H), lambda i: (i, 0)),
            ],
            out_specs=[pl.BlockSpec((Bp, _RB, H), lambda i: (0, i, 0))],
            core_axis_name=("c", "s"),
            dimension_semantics=(pltpu.PARALLEL,),
        )(x_hbm, pe_hbm, o_hbm)

    return pe_add_sc(x, pe_table)


def _tc_add_kernel(x_ref, pe_ref, o_ref):
    o_ref[...] = x_ref[...] + pe_ref[...][None, :, :]


def _tc_part(x, pe_table):
    Bp, S, H = x.shape
    return pl.pallas_call(
        _tc_add_kernel,
        grid=(S // _TC_BS,),
        in_specs=[
            pl.BlockSpec((Bp, _TC_BS, H), lambda i: (0, i, 0)),
            pl.BlockSpec((_TC_BS, H), lambda i: (i, 0)),
        ],
        out_specs=pl.BlockSpec((Bp, _TC_BS, H), lambda i: (0, i, 0)),
        out_shape=jax.ShapeDtypeStruct((Bp, S, H), x.dtype),
    )(x, pe_table)


def kernel(x, pe_table):
    B, S, H = x.shape
    split = B - _SC_BATCHES
    tc_out = _tc_part(x[:split], pe_table)
    sc_out = _sc_part(x[split:], pe_table)
    return jnp.concatenate([tc_out, sc_out], axis=0)


# speedup vs baseline: 2.1410x; 2.1410x over previous
import jax
import jax.numpy as jnp
from jax.experimental import pallas as pl
from jax.experimental.pallas import tpu as pltpu
from jax.experimental.pallas import tpu_sc as plsc

_RB = 4
_L = 16


def kernel(x, pe_table):
    B, S, H = x.shape

    mesh = plsc.VectorSubcoreMesh(core_axis_name="c", subcore_axis_name="s")

    @pl.kernel(out_type=jax.ShapeDtypeStruct((B, S, H), x.dtype), mesh=mesh)
    def pe_add_sc(x_hbm, pe_hbm, o_hbm):
        def body(x_vmem, pe_vmem, o_vmem):
            for r in range(_RB):

                @plsc.parallel_loop(0, H, step=_L, unroll=8)
                def _chunk(col, _r=r):
                    slc = pl.ds(col, _L)
                    pe_chunk = pe_vmem.at[_r].at[slc][...]
                    for b in range(B):
                        o_vmem.at[b].at[_r].at[slc][...] = (
                            x_vmem.at[b].at[_r].at[slc][...] + pe_chunk
                        )

        pltpu.emit_pipeline(
            body,
            grid=(S // _RB,),
            in_specs=[
                pl.BlockSpec((B, _RB, H), lambda i: (0, i, 0)),
                pl.BlockSpec((_RB, H), lambda i: (i, 0)),
            ],
            out_specs=[pl.BlockSpec((B, _RB, H), lambda i: (0, i, 0))],
            core_axis_name=("c", "s"),
            dimension_semantics=(pltpu.PARALLEL,),
            trace_scopes=False,
        )(x_hbm, pe_hbm, o_hbm)

    return pe_add_sc(x, pe_table)
